# baseline (speedup 1.0000x reference)
import functools

import jax
import jax.numpy as jnp
from jax import lax
from jax.experimental import pallas as pl
from jax.experimental.pallas import tpu as pltpu
from jax.experimental.pallas import tpu_sc as plsc

N = 10000
E = 160000
G = 128
D_IN = 128
D_H = 256
D_OUT = 128

NC = 2
NS = 16
L = 16

CHUNK = 128
CH_PER_TILE = 80
T_EDGES = CHUNK * CH_PER_TILE
E_PAD = T_EDGES * NS
ACC_ROWS = T_EDGES
ROWS_PER_TILE = N // NS
ACCZ_PER_TILE = ACC_ROWS // NS

P_CHUNK = 125
P_CH = 5
G_PER_TILE = G // NS


def _zero_vmem_2d(ref, rows, width):
  z = jnp.zeros((L,), jnp.float32)

  def body(r):
    for k in range(width // L):
      ref[r, pl.ds(k * L, L)] = z

  lax.fori_loop(0, rows, lambda r, _: (body(r), 0)[1], 0)


def _fill_ones_2d(ref, rows, width):
  o = jnp.ones((L,), jnp.float32)

  def body(r):
    for k in range(width // L):
      ref[r, pl.ds(k * L, L)] = o

  lax.fori_loop(0, rows, lambda r, _: (body(r), 0)[1], 0)


def _make_edge_agg(chunks, per_worker_dst):
  mesh = plsc.VectorSubcoreMesh(core_axis_name="c", subcore_axis_name="s")
  width = 128

  slab = 40
  n_loads = chunks // slab

  def body(table_hbm, srcw_hbm, dst_hbm, out_hbm, src_v, dst_v, rows0_v,
           rows1_v, acc_sh, gsem0, gsem1, ssem0, ssem1):
    c = lax.axis_index("c")
    s = lax.axis_index("s")
    w = c * NS + s
    dw = w if per_worker_dst else s

    _zero_vmem_2d(rows0_v, CHUNK, width)
    for b in range(ACCZ_PER_TILE // CHUNK):
      pltpu.sync_copy(rows0_v, acc_sh.at[pl.ds(s * ACCZ_PER_TILE + b * CHUNK,
                                               CHUNK)])
    plsc.subcore_barrier()

    def half(hf, carry):
      pltpu.sync_copy(srcw_hbm.at[w].at[pl.ds(hf * slab, slab)], src_v)
      pltpu.sync_copy(dst_hbm.at[dw].at[pl.ds(hf * slab, slab)], dst_v)
      pltpu.async_copy(table_hbm.at[src_v.at[0]], rows0_v, gsem0)
      pltpu.async_copy(table_hbm.at[src_v.at[1]], rows1_v, gsem1)

      def pair(j2, carry2):
        j = 2 * j2
        pltpu.make_async_copy(table_hbm.at[src_v.at[j]], rows0_v,
                              gsem0).wait()
        pltpu.sync_copy(rows0_v, acc_sh.at[dst_v.at[j]], add=True)

        @pl.when(j + 2 < slab)
        def _():
          pltpu.async_copy(table_hbm.at[src_v.at[j + 2]], rows0_v, gsem0)

        pltpu.make_async_copy(table_hbm.at[src_v.at[j + 1]], rows1_v,
                              gsem1).wait()
        pltpu.sync_copy(rows1_v, acc_sh.at[dst_v.at[j + 1]], add=True)

        @pl.when(j + 3 < slab)
        def _():
          pltpu.async_copy(table_hbm.at[src_v.at[j + 3]], rows1_v, gsem1)

        return carry2

      lax.fori_loop(0, slab // 2, pair, 0)
      return carry

    lax.fori_loop(0, n_loads, half, 0)
    plsc.subcore_barrier()

    pltpu.sync_copy(acc_sh.at[pl.ds(s * ACCZ_PER_TILE, ACCZ_PER_TILE)],
                    out_hbm.at[pl.ds(c * ACC_ROWS + s * ACCZ_PER_TILE,
                                     ACCZ_PER_TILE)])

  return pl.kernel(
      body,
      out_type=jax.ShapeDtypeStruct((2 * ACC_ROWS, width), jnp.float32),
      mesh=mesh,
      scratch_types=(
          pltpu.VMEM((slab, CHUNK), jnp.int32),
          pltpu.VMEM((slab, CHUNK), jnp.int32),
          pltpu.VMEM((CHUNK, width), jnp.float32),
          pltpu.VMEM((CHUNK, width), jnp.float32),
          pltpu.VMEM_SHARED((ACC_ROWS, width), jnp.float32),
          pltpu.SemaphoreType.DMA,
          pltpu.SemaphoreType.DMA,
          pltpu.SemaphoreType.DMA,
          pltpu.SemaphoreType.DMA,
      ),
  )


_edge_agg_l1 = _make_edge_agg(CH_PER_TILE // 2, True)
_edge_agg_h = _make_edge_agg(CH_PER_TILE, False)


def _deg_body(dst_hbm, deg_hbm, dst_v, ones_v, dacc_sh):
  c = lax.axis_index("c")
  s = lax.axis_index("s")
  w = c * NS + s

  pltpu.sync_copy(dst_hbm.at[w], dst_v)
  _zero_vmem_2d(ones_v, CHUNK, 128)
  for b in range(ACCZ_PER_TILE // CHUNK):
    pltpu.sync_copy(ones_v, dacc_sh.at[pl.ds(s * ACCZ_PER_TILE + b * CHUNK,
                                             CHUNK)])
  _fill_ones_2d(ones_v, CHUNK, 16)
  plsc.subcore_barrier()

  def chunk(j, carry):
    pltpu.sync_copy(ones_v, dacc_sh.at[dst_v.at[j]], add=True)
    return carry

  lax.fori_loop(0, CH_PER_TILE // 2, chunk, 0)
  plsc.subcore_barrier()
  pltpu.sync_copy(dacc_sh.at[pl.ds(s * ACCZ_PER_TILE, ACCZ_PER_TILE)],
                  deg_hbm.at[pl.ds(c * ACC_ROWS + s * ACCZ_PER_TILE,
                                   ACCZ_PER_TILE)])


_deg_k = pl.kernel(
    _deg_body,
    out_type=jax.ShapeDtypeStruct((2 * ACC_ROWS, 128), jnp.float32),
    mesh=plsc.VectorSubcoreMesh(core_axis_name="c", subcore_axis_name="s"),
    scratch_types=(
        pltpu.VMEM((CH_PER_TILE // 2, CHUNK), jnp.int32),
        pltpu.VMEM((CHUNK, 128), jnp.float32),
        pltpu.VMEM_SHARED((ACC_ROWS, 128), jnp.float32),
    ),
)


def _pool_body(h_hbm, batch_hbm, ridx_hbm, out_hbm, bidx_v, ridx_v, rows_v,
               acc_sh, gsem):
  c = lax.axis_index("c")
  s = lax.axis_index("s")
  w = c * NS + s

  pltpu.sync_copy(batch_hbm.at[s], bidx_v)
  pltpu.sync_copy(ridx_hbm.at[w], ridx_v)
  _zero_vmem_2d(rows_v, G_PER_TILE, D_H // NC)
  pltpu.sync_copy(rows_v.at[pl.ds(0, G_PER_TILE)],
                  acc_sh.at[pl.ds(s * G_PER_TILE, G_PER_TILE)])
  plsc.subcore_barrier()

  def chunk(j, carry):
    pltpu.async_copy(h_hbm.at[ridx_v.at[j]], rows_v, gsem).wait()
    pltpu.sync_copy(rows_v, acc_sh.at[bidx_v.at[j]], add=True)
    return carry

  lax.fori_loop(0, P_CH, chunk, 0)
  plsc.subcore_barrier()
  pltpu.sync_copy(acc_sh.at[pl.ds(s * G_PER_TILE, G_PER_TILE)],
                  out_hbm.at[pl.ds(c * G + s * G_PER_TILE, G_PER_TILE)])


_pool_k = pl.kernel(
    _pool_body,
    out_type=jax.ShapeDtypeStruct((2 * G, D_H // NC), jnp.float32),
    mesh=plsc.VectorSubcoreMesh(core_axis_name="c", subcore_axis_name="s"),
    scratch_types=(
        pltpu.VMEM((P_CH, P_CHUNK), jnp.int32),
        pltpu.VMEM((P_CH, P_CHUNK), jnp.int32),
        pltpu.VMEM((P_CHUNK, D_H // NC), jnp.float32),
        pltpu.VMEM_SHARED((G, D_H // NC), jnp.float32),
        pltpu.SemaphoreType.DMA,
    ),
)


_ROW_BLK = 400
_N_RB = N // _ROW_BLK


def _tc_root(x_parts, wr_parts):
  nx = len(x_parts)

  def body(*refs):
    x_refs = refs[:nx]
    wr_refs = refs[nx:2 * nx]
    out = refs[-1]
    out[...] = sum(jnp.dot(xr[...], wr[...],
                           preferred_element_type=jnp.float32)
                   for xr, wr in zip(x_refs, wr_refs))

  row_spec = lambda width: pl.BlockSpec((_ROW_BLK, width),
                                        lambda i, c: (i, 0))
  col_spec = lambda width: pl.BlockSpec((width, 128), lambda i, c: (0, c))
  return pl.pallas_call(
      body,
      grid=(_N_RB, 2),
      in_specs=([row_spec(xp.shape[1]) for xp in x_parts] +
                [col_spec(w.shape[0]) for w in wr_parts]),
      out_specs=pl.BlockSpec((_ROW_BLK, 128), lambda i, c: (c * _N_RB + i, 0)),
      out_shape=jax.ShapeDtypeStruct((2 * N, 128), jnp.float32),
  )(*x_parts, *wr_parts)


def _tc_combine(a_parts, wl_parts, root, d0, d1, b2d):
  na = len(a_parts)

  def body(*refs):
    a_refs = refs[:na]
    wl_refs = refs[na:2 * na]
    rootr, d0r, d1r, br = refs[-5], refs[-4], refs[-3], refs[-2]
    out = refs[-1]
    deg = d0r[:, 0:1] + d1r[:, 0:1]
    scale = 1.0 / jnp.maximum(deg, 1.0)
    m = sum(jnp.dot(a[...], w[...], preferred_element_type=jnp.float32)
            for a, w in zip(a_refs, wl_refs))
    brow = br[pl.ds(pl.program_id(1), 1), :]
    out[...] = jnp.maximum(m * scale + brow + rootr[...], 0.0)

  row_spec = lambda width: pl.BlockSpec((_ROW_BLK, width),
                                        lambda i, c: (i, 0))
  col_spec = lambda width: pl.BlockSpec((width, 128), lambda i, c: (0, c))
  return pl.pallas_call(
      body,
      grid=(_N_RB, 2),
      in_specs=([row_spec(a.shape[1]) for a in a_parts] +
                [col_spec(w.shape[0]) for w in wl_parts] +
                [pl.BlockSpec((_ROW_BLK, 128), lambda i, c: (c * _N_RB + i, 0)),
                 row_spec(d0.shape[1]), row_spec(d1.shape[1]),
                 pl.BlockSpec((2, 128), lambda i, c: (0, 0))]),
      out_specs=pl.BlockSpec((_ROW_BLK, 128), lambda i, c: (c * _N_RB + i, 0)),
      out_shape=jax.ShapeDtypeStruct((2 * N, 128), jnp.float32),
  )(*a_parts, *wl_parts, root, d0, d1, b2d)


def _tc_layer(a_parts, wl_parts, x_parts, wr_parts, d0, d1, b2d):
  na, nx = len(a_parts), len(x_parts)

  def body(*refs):
    a_refs = refs[:na]
    wl_refs = refs[na:2 * na]
    x_refs = refs[2 * na:2 * na + nx]
    wr_refs = refs[2 * na + nx:2 * na + 2 * nx]
    d0r, d1r, br = refs[-4], refs[-3], refs[-2]
    out = refs[-1]
    deg = d0r[:, 0:1] + d1r[:, 0:1]
    scale = 1.0 / jnp.maximum(deg, 1.0)
    m = sum(jnp.dot(a[...], w[...], preferred_element_type=jnp.float32)
            for a, w in zip(a_refs, wl_refs))
    r = sum(jnp.dot(x[...], w[...], preferred_element_type=jnp.float32)
            for x, w in zip(x_refs, wr_refs))
    brow = br[pl.ds(pl.program_id(1), 1), :]
    out[...] = jnp.maximum(m * scale + brow + r, 0.0)

  row_spec = lambda width: pl.BlockSpec((_ROW_BLK, width),
                                        lambda i, c: (i, 0))
  col_spec = lambda width: pl.BlockSpec((width, 128), lambda i, c: (0, c))
  in_specs = (
      [row_spec(a.shape[1]) for a in a_parts] +
      [col_spec(w.shape[0]) for w in wl_parts] +
      [row_spec(x.shape[1]) for x in x_parts] +
      [col_spec(w.shape[0]) for w in wr_parts] +
      [row_spec(d0.shape[1]), row_spec(d1.shape[1]),
       pl.BlockSpec((2, 128), lambda i, c: (0, 0))]
  )
  return pl.pallas_call(
      body,
      grid=(_N_RB, 2),
      in_specs=in_specs,
      out_specs=pl.BlockSpec((_ROW_BLK, 128), lambda i, c: (c * _N_RB + i, 0)),
      out_shape=jax.ShapeDtypeStruct((2 * N, 128), jnp.float32),
  )(*a_parts, *wl_parts, *x_parts, *wr_parts, d0, d1, b2d)


def _head_body(p0, p1, w1a, w1b, b1, w2, b2, out):
  h = (jnp.dot(p0[...], w1a[...], preferred_element_type=jnp.float32) +
       jnp.dot(p1[...], w1b[...], preferred_element_type=jnp.float32) +
       b1[...])
  h = jnp.maximum(h, 0.0)
  out[...] = jnp.dot(h, w2[...], preferred_element_type=jnp.float32) + b2[...]


def _tc_head(p0, p1, w1a, w1b, b1, w2, b2):
  return pl.pallas_call(
      _head_body,
      out_shape=jax.ShapeDtypeStruct((G, D_OUT), jnp.float32),
  )(p0, p1, w1a, w1b, b1, w2, b2)


def kernel(x, edge_index, batch, Wl1, bl1, Wr1, Wl2, bl2, Wr2, Wl3, bl3, Wr3,
           W_lin1, b_lin1, W_lin2, b_lin2):
  src = edge_index[0].astype(jnp.int32)
  dst = edge_index[1].astype(jnp.int32)

  pad = E_PAD - E
  src_p = jnp.concatenate([src, jnp.zeros((pad,), jnp.int32)])
  dst_p = jnp.concatenate([dst, jnp.full((pad,), N, jnp.int32)])
  src_l1 = src_p.reshape(NC * NS, CH_PER_TILE // 2, CHUNK)
  src_l1 = jnp.concatenate([src_l1[:NS], src_l1[NS:] + N], axis=0)
  dst_l1 = dst_p.reshape(NC * NS, CH_PER_TILE // 2, CHUNK)
  src_r = src_p.reshape(NS, CH_PER_TILE, CHUNK)
  srcw = jnp.concatenate([src_r, src_r + N], axis=0)
  dst_r = dst_p.reshape(NS, CH_PER_TILE, CHUNK)
  batch_r = batch.astype(jnp.int32).reshape(NS, P_CH, P_CHUNK)
  ridx = jnp.arange(N, dtype=jnp.int32).reshape(NS, P_CH, P_CHUNK)
  ridx_w = jnp.concatenate([ridx, ridx + N], axis=0)

  agg1 = _edge_agg_l1(jnp.concatenate([x, x], axis=0), src_l1, dst_l1)
  degp = _deg_k(dst_l1)
  d0, d1 = degp[:N], degp[ACC_ROWS:ACC_ROWS + N]
  root = _tc_root([x], [Wr1.T])
  h = _tc_combine([agg1[:N], agg1[ACC_ROWS:ACC_ROWS + N]], [Wl1.T, Wl1.T],
                  root, d0, d1, bl1.reshape(2, 128))

  for Wl, bl, Wr in ((Wl2, bl2, Wr2), (Wl3, bl3, Wr3)):
    agg = _edge_agg_h(h, srcw, dst_r)
    root = _tc_root([h[:N], h[N:]], [Wr.T[:128], Wr.T[128:]])
    h = _tc_combine([agg[:N], agg[ACC_ROWS:ACC_ROWS + N]],
                    [Wl.T[:128], Wl.T[128:]],
                    root, d0, d1, bl.reshape(2, 128))

  pooled = _pool_k(h, batch_r, ridx_w)
  out = _tc_head(pooled[:G], pooled[G:],
                 W_lin1.T[:128], W_lin1.T[128:], b_lin1.reshape(1, D_H),
                 W_lin2.T, b_lin2.reshape(1, D_OUT))
  return out


# speedup vs baseline: 1.0679x; 1.0679x over previous
import functools

import jax
import jax.numpy as jnp
from jax import lax
from jax.experimental import pallas as pl
from jax.experimental.pallas import tpu as pltpu
from jax.experimental.pallas import tpu_sc as plsc

N = 10000
E = 160000
G = 128
D_IN = 128
D_H = 256
D_OUT = 128

NC = 2
NS = 16
L = 16

CHUNK = 128
CH_PER_TILE = 80
T_EDGES = CHUNK * CH_PER_TILE
E_PAD = T_EDGES * NS
ACC_ROWS = T_EDGES
ROWS_PER_TILE = N // NS
ACCZ_PER_TILE = ACC_ROWS // NS

P_CHUNK = 125
P_CH = 5
G_PER_TILE = G // NS


def _zero_vmem_2d(ref, rows, width):
  z = jnp.zeros((L,), jnp.float32)

  def body(r):
    for k in range(width // L):
      ref[r, pl.ds(k * L, L)] = z

  lax.fori_loop(0, rows, lambda r, _: (body(r), 0)[1], 0)


def _fill_ones_2d(ref, rows, width):
  o = jnp.ones((L,), jnp.float32)

  def body(r):
    for k in range(width // L):
      ref[r, pl.ds(k * L, L)] = o

  lax.fori_loop(0, rows, lambda r, _: (body(r), 0)[1], 0)


def _make_edge_agg(chunks, per_worker_dst):
  mesh = plsc.VectorSubcoreMesh(core_axis_name="c", subcore_axis_name="s")
  width = 128

  slab = 40
  n_loads = chunks // slab

  def body(table_hbm, srcw_hbm, dst_hbm, out_hbm, src_v, dst_v, rows0_v,
           rows1_v, acc_sh, gsem0, gsem1, ssem0, ssem1):
    c = lax.axis_index("c")
    s = lax.axis_index("s")
    w = c * NS + s
    dw = w if per_worker_dst else s

    _zero_vmem_2d(rows0_v, CHUNK, width)
    for b in range(ACCZ_PER_TILE // CHUNK):
      pltpu.sync_copy(rows0_v, acc_sh.at[pl.ds(s * ACCZ_PER_TILE + b * CHUNK,
                                               CHUNK)])
    plsc.subcore_barrier()

    def half(hf, carry):
      pltpu.sync_copy(srcw_hbm.at[w].at[pl.ds(hf * slab, slab)], src_v)
      pltpu.sync_copy(dst_hbm.at[dw].at[pl.ds(hf * slab, slab)], dst_v)
      pltpu.async_copy(table_hbm.at[src_v.at[0]], rows0_v, gsem0)
      pltpu.async_copy(table_hbm.at[src_v.at[1]], rows1_v, gsem1)

      def pair(j2, carry2):
        j = 2 * j2
        pltpu.make_async_copy(table_hbm.at[src_v.at[j]], rows0_v,
                              gsem0).wait()
        pltpu.sync_copy(rows0_v, acc_sh.at[dst_v.at[j]], add=True)

        @pl.when(j + 2 < slab)
        def _():
          pltpu.async_copy(table_hbm.at[src_v.at[j + 2]], rows0_v, gsem0)

        pltpu.make_async_copy(table_hbm.at[src_v.at[j + 1]], rows1_v,
                              gsem1).wait()
        pltpu.sync_copy(rows1_v, acc_sh.at[dst_v.at[j + 1]], add=True)

        @pl.when(j + 3 < slab)
        def _():
          pltpu.async_copy(table_hbm.at[src_v.at[j + 3]], rows1_v, gsem1)

        return carry2

      lax.fori_loop(0, slab // 2, pair, 0)
      return carry

    lax.fori_loop(0, n_loads, half, 0)
    plsc.subcore_barrier()

    pltpu.sync_copy(acc_sh.at[pl.ds(s * ACCZ_PER_TILE, ACCZ_PER_TILE)],
                    out_hbm.at[pl.ds(c * ACC_ROWS + s * ACCZ_PER_TILE,
                                     ACCZ_PER_TILE)])

  return pl.kernel(
      body,
      out_type=jax.ShapeDtypeStruct((2 * ACC_ROWS, width), jnp.float32),
      mesh=mesh,
      scratch_types=(
          pltpu.VMEM((slab, CHUNK), jnp.int32),
          pltpu.VMEM((slab, CHUNK), jnp.int32),
          pltpu.VMEM((CHUNK, width), jnp.float32),
          pltpu.VMEM((CHUNK, width), jnp.float32),
          pltpu.VMEM_SHARED((ACC_ROWS, width), jnp.float32),
          pltpu.SemaphoreType.DMA,
          pltpu.SemaphoreType.DMA,
          pltpu.SemaphoreType.DMA,
          pltpu.SemaphoreType.DMA,
      ),
  )


_edge_agg_l1 = _make_edge_agg(CH_PER_TILE // 2, True)
_edge_agg_h = _make_edge_agg(CH_PER_TILE, False)


def _deg_body(dst_hbm, deg_hbm, dst_v, ones_v, dacc_sh):
  c = lax.axis_index("c")
  s = lax.axis_index("s")
  w = c * NS + s

  pltpu.sync_copy(dst_hbm.at[w], dst_v)
  _zero_vmem_2d(ones_v, CHUNK, 128)
  for b in range(ACCZ_PER_TILE // CHUNK):
    pltpu.sync_copy(ones_v, dacc_sh.at[pl.ds(s * ACCZ_PER_TILE + b * CHUNK,
                                             CHUNK)])
  _fill_ones_2d(ones_v, CHUNK, 16)
  plsc.subcore_barrier()

  def chunk(j, carry):
    pltpu.sync_copy(ones_v, dacc_sh.at[dst_v.at[j]], add=True)
    return carry

  lax.fori_loop(0, CH_PER_TILE // 2, chunk, 0)
  plsc.subcore_barrier()
  pltpu.sync_copy(dacc_sh.at[pl.ds(s * ACCZ_PER_TILE, ACCZ_PER_TILE)],
                  deg_hbm.at[pl.ds(c * ACC_ROWS + s * ACCZ_PER_TILE,
                                   ACCZ_PER_TILE)])


_deg_k = pl.kernel(
    _deg_body,
    out_type=jax.ShapeDtypeStruct((2 * ACC_ROWS, 128), jnp.float32),
    mesh=plsc.VectorSubcoreMesh(core_axis_name="c", subcore_axis_name="s"),
    scratch_types=(
        pltpu.VMEM((CH_PER_TILE // 2, CHUNK), jnp.int32),
        pltpu.VMEM((CHUNK, 128), jnp.float32),
        pltpu.VMEM_SHARED((ACC_ROWS, 128), jnp.float32),
    ),
)


def _pool_body(h_hbm, batch_hbm, ridx_hbm, out_hbm, bidx_v, ridx_v, rows_v,
               acc_sh, gsem):
  c = lax.axis_index("c")
  s = lax.axis_index("s")
  w = c * NS + s

  pltpu.sync_copy(batch_hbm.at[s], bidx_v)
  pltpu.sync_copy(ridx_hbm.at[w], ridx_v)
  _zero_vmem_2d(rows_v, G_PER_TILE, D_H // NC)
  pltpu.sync_copy(rows_v.at[pl.ds(0, G_PER_TILE)],
                  acc_sh.at[pl.ds(s * G_PER_TILE, G_PER_TILE)])
  plsc.subcore_barrier()

  def chunk(j, carry):
    pltpu.async_copy(h_hbm.at[ridx_v.at[j]], rows_v, gsem).wait()
    pltpu.sync_copy(rows_v, acc_sh.at[bidx_v.at[j]], add=True)
    return carry

  lax.fori_loop(0, P_CH, chunk, 0)
  plsc.subcore_barrier()
  pltpu.sync_copy(acc_sh.at[pl.ds(s * G_PER_TILE, G_PER_TILE)],
                  out_hbm.at[pl.ds(c * G + s * G_PER_TILE, G_PER_TILE)])


_pool_k = pl.kernel(
    _pool_body,
    out_type=jax.ShapeDtypeStruct((2 * G, D_H // NC), jnp.float32),
    mesh=plsc.VectorSubcoreMesh(core_axis_name="c", subcore_axis_name="s"),
    scratch_types=(
        pltpu.VMEM((P_CH, P_CHUNK), jnp.int32),
        pltpu.VMEM((P_CH, P_CHUNK), jnp.int32),
        pltpu.VMEM((P_CHUNK, D_H // NC), jnp.float32),
        pltpu.VMEM_SHARED((G, D_H // NC), jnp.float32),
        pltpu.SemaphoreType.DMA,
    ),
)


_ROW_BLK = 1000
_N_RB = N // _ROW_BLK


def _tc_layer(a_parts, wl_parts, x_parts, wr_parts, d0, d1, b2d):
  na, nx = len(a_parts), len(x_parts)

  def body(*refs):
    a_refs = refs[:na]
    wl_refs = refs[na:2 * na]
    x_refs = refs[2 * na:2 * na + nx]
    wr_refs = refs[2 * na + nx:2 * na + 2 * nx]
    d0r, d1r, br = refs[-4], refs[-3], refs[-2]
    out = refs[-1]
    deg = d0r[:, 0:1] + d1r[:, 0:1]
    scale = 1.0 / jnp.maximum(deg, 1.0)
    m = sum(jnp.dot(a[...], w[...], preferred_element_type=jnp.float32)
            for a, w in zip(a_refs, wl_refs))
    r = sum(jnp.dot(x[...], w[...], preferred_element_type=jnp.float32)
            for x, w in zip(x_refs, wr_refs))
    brow = br[pl.ds(pl.program_id(1), 1), :]
    out[...] = jnp.maximum(m * scale + brow + r, 0.0)

  row_spec = lambda width: pl.BlockSpec((_ROW_BLK, width),
                                        lambda i, c: (i, 0))
  col_spec = lambda width: pl.BlockSpec((width, 128), lambda i, c: (0, c))
  in_specs = (
      [row_spec(a.shape[1]) for a in a_parts] +
      [col_spec(w.shape[0]) for w in wl_parts] +
      [row_spec(x.shape[1]) for x in x_parts] +
      [col_spec(w.shape[0]) for w in wr_parts] +
      [row_spec(d0.shape[1]), row_spec(d1.shape[1]),
       pl.BlockSpec((2, 128), lambda i, c: (0, 0))]
  )
  return pl.pallas_call(
      body,
      grid=(_N_RB, 2),
      in_specs=in_specs,
      out_specs=pl.BlockSpec((_ROW_BLK, 128), lambda i, c: (c * _N_RB + i, 0)),
      out_shape=jax.ShapeDtypeStruct((2 * N, 128), jnp.float32),
  )(*a_parts, *wl_parts, *x_parts, *wr_parts, d0, d1, b2d)


def _head_body(p0, p1, w1a, w1b, b1, w2, b2, out):
  h = (jnp.dot(p0[...], w1a[...], preferred_element_type=jnp.float32) +
       jnp.dot(p1[...], w1b[...], preferred_element_type=jnp.float32) +
       b1[...])
  h = jnp.maximum(h, 0.0)
  out[...] = jnp.dot(h, w2[...], preferred_element_type=jnp.float32) + b2[...]


def _tc_head(p0, p1, w1a, w1b, b1, w2, b2):
  return pl.pallas_call(
      _head_body,
      out_shape=jax.ShapeDtypeStruct((G, D_OUT), jnp.float32),
  )(p0, p1, w1a, w1b, b1, w2, b2)


def kernel(x, edge_index, batch, Wl1, bl1, Wr1, Wl2, bl2, Wr2, Wl3, bl3, Wr3,
           W_lin1, b_lin1, W_lin2, b_lin2):
  src = edge_index[0].astype(jnp.int32)
  dst = edge_index[1].astype(jnp.int32)

  pad = E_PAD - E
  src_p = jnp.concatenate([src, jnp.zeros((pad,), jnp.int32)])
  dst_p = jnp.concatenate([dst, jnp.full((pad,), N, jnp.int32)])
  src_l1 = src_p.reshape(NC * NS, CH_PER_TILE // 2, CHUNK)
  src_l1 = jnp.concatenate([src_l1[:NS], src_l1[NS:] + N], axis=0)
  dst_l1 = dst_p.reshape(NC * NS, CH_PER_TILE // 2, CHUNK)
  src_r = src_p.reshape(NS, CH_PER_TILE, CHUNK)
  srcw = jnp.concatenate([src_r, src_r + N], axis=0)
  dst_r = dst_p.reshape(NS, CH_PER_TILE, CHUNK)
  batch_r = batch.astype(jnp.int32).reshape(NS, P_CH, P_CHUNK)
  ridx = jnp.arange(N, dtype=jnp.int32).reshape(NS, P_CH, P_CHUNK)
  ridx_w = jnp.concatenate([ridx, ridx + N], axis=0)

  agg1 = _edge_agg_l1(jnp.concatenate([x, x], axis=0), src_l1, dst_l1)
  degp = _deg_k(dst_l1)
  d0, d1 = degp[:N], degp[ACC_ROWS:ACC_ROWS + N]
  h = _tc_layer([agg1[:N], agg1[ACC_ROWS:ACC_ROWS + N]], [Wl1.T, Wl1.T],
                [x], [Wr1.T], d0, d1, bl1.reshape(2, 128))

  for Wl, bl, Wr in ((Wl2, bl2, Wr2), (Wl3, bl3, Wr3)):
    agg = _edge_agg_h(h, srcw, dst_r)
    h = _tc_layer([agg[:N], agg[ACC_ROWS:ACC_ROWS + N]],
                  [Wl.T[:128], Wl.T[128:]],
                  [h[:N], h[N:]],
                  [Wr.T[:128], Wr.T[128:]],
                  d0, d1, bl.reshape(2, 128))

  pooled = _pool_k(h, batch_r, ridx_w)
  out = _tc_head(pooled[:G], pooled[G:],
                 W_lin1.T[:128], W_lin1.T[128:], b_lin1.reshape(1, D_H),
                 W_lin2.T, b_lin2.reshape(1, D_OUT))
  return out


# speedup vs baseline: 1.0945x; 1.0249x over previous
import functools

import jax
import jax.numpy as jnp
from jax import lax
from jax.experimental import pallas as pl
from jax.experimental.pallas import tpu as pltpu
from jax.experimental.pallas import tpu_sc as plsc

N = 10000
E = 160000
G = 128
D_IN = 128
D_H = 256
D_OUT = 128

NC = 2
NS = 16
L = 16

CHUNK = 128
CH_PER_TILE = 80
T_EDGES = CHUNK * CH_PER_TILE
E_PAD = T_EDGES * NS
ACC_ROWS = T_EDGES
ROWS_PER_TILE = N // NS
ACCZ_PER_TILE = ACC_ROWS // NS

P_CHUNK = 125
P_CH = 5
G_PER_TILE = G // NS


def _zero_vmem_2d(ref, rows, width):
  z = jnp.zeros((L,), jnp.float32)

  def body(r):
    for k in range(width // L):
      ref[r, pl.ds(k * L, L)] = z

  lax.fori_loop(0, rows, lambda r, _: (body(r), 0)[1], 0)


def _fill_ones_2d(ref, rows, width):
  o = jnp.ones((L,), jnp.float32)

  def body(r):
    for k in range(width // L):
      ref[r, pl.ds(k * L, L)] = o

  lax.fori_loop(0, rows, lambda r, _: (body(r), 0)[1], 0)


def _make_edge_agg(chunks, per_worker_dst):
  mesh = plsc.VectorSubcoreMesh(core_axis_name="c", subcore_axis_name="s")
  width = 128

  slab = 40
  n_loads = chunks // slab

  def body(table_hbm, srcw_hbm, dst_hbm, out_hbm, src_v, dst_v, rows0_v,
           rows1_v, acc_sh, gsem0, gsem1, ssem0, ssem1):
    c = lax.axis_index("c")
    s = lax.axis_index("s")
    w = c * NS + s
    dw = w if per_worker_dst else s

    _zero_vmem_2d(rows0_v, CHUNK, width)
    for b in range(ACCZ_PER_TILE // CHUNK):
      pltpu.sync_copy(rows0_v, acc_sh.at[pl.ds(s * ACCZ_PER_TILE + b * CHUNK,
                                               CHUNK)])
    plsc.subcore_barrier()

    def half(hf, carry):
      pltpu.sync_copy(srcw_hbm.at[w].at[pl.ds(hf * slab, slab)], src_v)
      pltpu.sync_copy(dst_hbm.at[dw].at[pl.ds(hf * slab, slab)], dst_v)
      pltpu.async_copy(table_hbm.at[src_v.at[0]], rows0_v, gsem0)
      pltpu.async_copy(table_hbm.at[src_v.at[1]], rows1_v, gsem1)

      def pair(j2, carry2):
        j = 2 * j2
        pltpu.make_async_copy(table_hbm.at[src_v.at[j]], rows0_v,
                              gsem0).wait()
        pltpu.sync_copy(rows0_v, acc_sh.at[dst_v.at[j]], add=True)

        @pl.when(j + 2 < slab)
        def _():
          pltpu.async_copy(table_hbm.at[src_v.at[j + 2]], rows0_v, gsem0)

        pltpu.make_async_copy(table_hbm.at[src_v.at[j + 1]], rows1_v,
                              gsem1).wait()
        pltpu.sync_copy(rows1_v, acc_sh.at[dst_v.at[j + 1]], add=True)

        @pl.when(j + 3 < slab)
        def _():
          pltpu.async_copy(table_hbm.at[src_v.at[j + 3]], rows1_v, gsem1)

        return carry2

      lax.fori_loop(0, slab // 2, pair, 0)
      return carry

    lax.fori_loop(0, n_loads, half, 0)
    plsc.subcore_barrier()

    pltpu.sync_copy(acc_sh.at[pl.ds(s * ACCZ_PER_TILE, ACCZ_PER_TILE)],
                    out_hbm.at[pl.ds(c * ACC_ROWS + s * ACCZ_PER_TILE,
                                     ACCZ_PER_TILE)])

  return pl.kernel(
      body,
      out_type=jax.ShapeDtypeStruct((2 * ACC_ROWS, width), jnp.float32),
      mesh=mesh,
      scratch_types=(
          pltpu.VMEM((slab, CHUNK), jnp.int32),
          pltpu.VMEM((slab, CHUNK), jnp.int32),
          pltpu.VMEM((CHUNK, width), jnp.float32),
          pltpu.VMEM((CHUNK, width), jnp.float32),
          pltpu.VMEM_SHARED((ACC_ROWS, width), jnp.float32),
          pltpu.SemaphoreType.DMA,
          pltpu.SemaphoreType.DMA,
          pltpu.SemaphoreType.DMA,
          pltpu.SemaphoreType.DMA,
      ),
  )


_edge_agg_l1 = _make_edge_agg(CH_PER_TILE // 2, True)
_edge_agg_h = _make_edge_agg(CH_PER_TILE, False)


def _deg_body(dst_hbm, deg_hbm, dst_v, ones_v, dacc_sh):
  c = lax.axis_index("c")
  s = lax.axis_index("s")
  w = c * NS + s

  pltpu.sync_copy(dst_hbm.at[w], dst_v)
  _zero_vmem_2d(ones_v, CHUNK, 128)
  for b in range(ACCZ_PER_TILE // CHUNK):
    pltpu.sync_copy(ones_v, dacc_sh.at[pl.ds(s * ACCZ_PER_TILE + b * CHUNK,
                                             CHUNK)])
  _fill_ones_2d(ones_v, CHUNK, 16)
  plsc.subcore_barrier()

  def chunk(j, carry):
    pltpu.sync_copy(ones_v, dacc_sh.at[dst_v.at[j]], add=True)
    return carry

  lax.fori_loop(0, CH_PER_TILE // 2, chunk, 0)
  plsc.subcore_barrier()
  pltpu.sync_copy(dacc_sh.at[pl.ds(s * ACCZ_PER_TILE, ACCZ_PER_TILE)],
                  deg_hbm.at[pl.ds(c * ACC_ROWS + s * ACCZ_PER_TILE,
                                   ACCZ_PER_TILE)])


_deg_k = pl.kernel(
    _deg_body,
    out_type=jax.ShapeDtypeStruct((2 * ACC_ROWS, 128), jnp.float32),
    mesh=plsc.VectorSubcoreMesh(core_axis_name="c", subcore_axis_name="s"),
    scratch_types=(
        pltpu.VMEM((CH_PER_TILE // 2, CHUNK), jnp.int32),
        pltpu.VMEM((CHUNK, 128), jnp.float32),
        pltpu.VMEM_SHARED((ACC_ROWS, 128), jnp.float32),
    ),
)


def _pool_body(h_hbm, batch_hbm, ridx_hbm, out_hbm, bidx_v, ridx_v, rows_v,
               acc_sh, gsem):
  c = lax.axis_index("c")
  s = lax.axis_index("s")
  w = c * NS + s

  pltpu.sync_copy(batch_hbm.at[s], bidx_v)
  pltpu.sync_copy(ridx_hbm.at[w], ridx_v)
  _zero_vmem_2d(rows_v, G_PER_TILE, D_H // NC)
  pltpu.sync_copy(rows_v.at[pl.ds(0, G_PER_TILE)],
                  acc_sh.at[pl.ds(s * G_PER_TILE, G_PER_TILE)])
  plsc.subcore_barrier()

  def chunk(j, carry):
    pltpu.async_copy(h_hbm.at[ridx_v.at[j]], rows_v, gsem).wait()
    pltpu.sync_copy(rows_v, acc_sh.at[bidx_v.at[j]], add=True)
    return carry

  lax.fori_loop(0, P_CH, chunk, 0)
  plsc.subcore_barrier()
  pltpu.sync_copy(acc_sh.at[pl.ds(s * G_PER_TILE, G_PER_TILE)],
                  out_hbm.at[pl.ds(c * G + s * G_PER_TILE, G_PER_TILE)])


_pool_k = pl.kernel(
    _pool_body,
    out_type=jax.ShapeDtypeStruct((2 * G, D_H // NC), jnp.float32),
    mesh=plsc.VectorSubcoreMesh(core_axis_name="c", subcore_axis_name="s"),
    scratch_types=(
        pltpu.VMEM((P_CH, P_CHUNK), jnp.int32),
        pltpu.VMEM((P_CH, P_CHUNK), jnp.int32),
        pltpu.VMEM((P_CHUNK, D_H // NC), jnp.float32),
        pltpu.VMEM_SHARED((G, D_H // NC), jnp.float32),
        pltpu.SemaphoreType.DMA,
    ),
)


_ROW_BLK = 2000
_N_RB = N // _ROW_BLK


def _tc_layer(a_parts, wl_parts, x_parts, wr_parts, d0, d1, b2d):
  na, nx = len(a_parts), len(x_parts)

  def body(*refs):
    a_refs = refs[:na]
    wl_refs = refs[na:2 * na]
    x_refs = refs[2 * na:2 * na + nx]
    wr_refs = refs[2 * na + nx:2 * na + 2 * nx]
    d0r, d1r, br = refs[-4], refs[-3], refs[-2]
    out = refs[-1]
    deg = d0r[:, 0:1] + d1r[:, 0:1]
    scale = 1.0 / jnp.maximum(deg, 1.0)
    m = sum(jnp.dot(a[...], w[...], preferred_element_type=jnp.float32)
            for a, w in zip(a_refs, wl_refs))
    r = sum(jnp.dot(x[...], w[...], preferred_element_type=jnp.float32)
            for x, w in zip(x_refs, wr_refs))
    brow = br[pl.ds(pl.program_id(1), 1), :]
    out[...] = jnp.maximum(m * scale + brow + r, 0.0)

  row_spec = lambda width: pl.BlockSpec((_ROW_BLK, width),
                                        lambda i, c: (i, 0))
  col_spec = lambda width: pl.BlockSpec((width, 128), lambda i, c: (0, c))
  in_specs = (
      [row_spec(a.shape[1]) for a in a_parts] +
      [col_spec(w.shape[0]) for w in wl_parts] +
      [row_spec(x.shape[1]) for x in x_parts] +
      [col_spec(w.shape[0]) for w in wr_parts] +
      [row_spec(d0.shape[1]), row_spec(d1.shape[1]),
       pl.BlockSpec((2, 128), lambda i, c: (0, 0))]
  )
  return pl.pallas_call(
      body,
      grid=(_N_RB, 2),
      in_specs=in_specs,
      out_specs=pl.BlockSpec((_ROW_BLK, 128), lambda i, c: (c * _N_RB + i, 0)),
      out_shape=jax.ShapeDtypeStruct((2 * N, 128), jnp.float32),
  )(*a_parts, *wl_parts, *x_parts, *wr_parts, d0, d1, b2d)


def _head_body(p0, p1, w1a, w1b, b1, w2, b2, out):
  h = (jnp.dot(p0[...], w1a[...], preferred_element_type=jnp.float32) +
       jnp.dot(p1[...], w1b[...], preferred_element_type=jnp.float32) +
       b1[...])
  h = jnp.maximum(h, 0.0)
  out[...] = jnp.dot(h, w2[...], preferred_element_type=jnp.float32) + b2[...]


def _tc_head(p0, p1, w1a, w1b, b1, w2, b2):
  return pl.pallas_call(
      _head_body,
      out_shape=jax.ShapeDtypeStruct((G, D_OUT), jnp.float32),
  )(p0, p1, w1a, w1b, b1, w2, b2)


def kernel(x, edge_index, batch, Wl1, bl1, Wr1, Wl2, bl2, Wr2, Wl3, bl3, Wr3,
           W_lin1, b_lin1, W_lin2, b_lin2):
  src = edge_index[0].astype(jnp.int32)
  dst = edge_index[1].astype(jnp.int32)

  pad = E_PAD - E
  src_p = jnp.concatenate([src, jnp.zeros((pad,), jnp.int32)])
  dst_p = jnp.concatenate([dst, jnp.full((pad,), N, jnp.int32)])
  src_l1 = src_p.reshape(NC * NS, CH_PER_TILE // 2, CHUNK)
  src_l1 = jnp.concatenate([src_l1[:NS], src_l1[NS:] + N], axis=0)
  dst_l1 = dst_p.reshape(NC * NS, CH_PER_TILE // 2, CHUNK)
  src_r = src_p.reshape(NS, CH_PER_TILE, CHUNK)
  srcw = jnp.concatenate([src_r, src_r + N], axis=0)
  dst_r = dst_p.reshape(NS, CH_PER_TILE, CHUNK)
  batch_r = batch.astype(jnp.int32).reshape(NS, P_CH, P_CHUNK)
  ridx = jnp.arange(N, dtype=jnp.int32).reshape(NS, P_CH, P_CHUNK)
  ridx_w = jnp.concatenate([ridx, ridx + N], axis=0)

  agg1 = _edge_agg_l1(jnp.concatenate([x, x], axis=0), src_l1, dst_l1)
  degp = _deg_k(dst_l1)
  d0, d1 = degp[:N], degp[ACC_ROWS:ACC_ROWS + N]
  h = _tc_layer([agg1[:N], agg1[ACC_ROWS:ACC_ROWS + N]], [Wl1.T, Wl1.T],
                [x], [Wr1.T], d0, d1, bl1.reshape(2, 128))

  for Wl, bl, Wr in ((Wl2, bl2, Wr2), (Wl3, bl3, Wr3)):
    agg = _edge_agg_h(h, srcw, dst_r)
    h = _tc_layer([agg[:N], agg[ACC_ROWS:ACC_ROWS + N]],
                  [Wl.T[:128], Wl.T[128:]],
                  [h[:N], h[N:]],
                  [Wr.T[:128], Wr.T[128:]],
                  d0, d1, bl.reshape(2, 128))

  pooled = _pool_k(h, batch_r, ridx_w)
  out = _tc_head(pooled[:G], pooled[G:],
                 W_lin1.T[:128], W_lin1.T[128:], b_lin1.reshape(1, D_H),
                 W_lin2.T, b_lin2.reshape(1, D_OUT))
  return out


# speedup vs baseline: 1.0961x; 1.0015x over previous
import functools

import jax
import jax.numpy as jnp
from jax import lax
from jax.experimental import pallas as pl
from jax.experimental.pallas import tpu as pltpu
from jax.experimental.pallas import tpu_sc as plsc

N = 10000
E = 160000
G = 128
D_IN = 128
D_H = 256
D_OUT = 128

NC = 2
NS = 16
L = 16

CHUNK = 128
CH_PER_TILE = 80
T_EDGES = CHUNK * CH_PER_TILE
E_PAD = T_EDGES * NS
ACC_ROWS = T_EDGES
ROWS_PER_TILE = N // NS
ACCZ_PER_TILE = ACC_ROWS // NS

P_CHUNK = 125
P_CH = 5
G_PER_TILE = G // NS


def _zero_vmem_2d(ref, rows, width):
  z = jnp.zeros((L,), jnp.float32)

  def body(r):
    for k in range(width // L):
      ref[r, pl.ds(k * L, L)] = z

  lax.fori_loop(0, rows, lambda r, _: (body(r), 0)[1], 0)


def _fill_ones_2d(ref, rows, width):
  o = jnp.ones((L,), jnp.float32)

  def body(r):
    for k in range(width // L):
      ref[r, pl.ds(k * L, L)] = o

  lax.fori_loop(0, rows, lambda r, _: (body(r), 0)[1], 0)


def _make_edge_agg(chunks, per_worker_dst):
  mesh = plsc.VectorSubcoreMesh(core_axis_name="c", subcore_axis_name="s")
  width = 128

  slab = 40
  n_loads = chunks // slab

  def body(table_hbm, srcw_hbm, dst_hbm, out_hbm, src_v, dst_v, rows0_v,
           rows1_v, acc_sh, gsem0, gsem1, ssem0, ssem1):
    c = lax.axis_index("c")
    s = lax.axis_index("s")
    w = c * NS + s
    dw = w if per_worker_dst else s

    _zero_vmem_2d(rows0_v, CHUNK, width)
    for b in range(ACCZ_PER_TILE // CHUNK):
      pltpu.sync_copy(rows0_v, acc_sh.at[pl.ds(s * ACCZ_PER_TILE + b * CHUNK,
                                               CHUNK)])
    plsc.subcore_barrier()

    def half(hf, carry):
      pltpu.sync_copy(srcw_hbm.at[w].at[pl.ds(hf * slab, slab)], src_v)
      pltpu.sync_copy(dst_hbm.at[dw].at[pl.ds(hf * slab, slab)], dst_v)
      pltpu.async_copy(table_hbm.at[src_v.at[0]], rows0_v, gsem0)
      pltpu.async_copy(table_hbm.at[src_v.at[1]], rows1_v, gsem1)

      def pair(j2, carry2):
        j = 2 * j2
        pltpu.make_async_copy(table_hbm.at[src_v.at[j]], rows0_v,
                              gsem0).wait()
        pltpu.sync_copy(rows0_v, acc_sh.at[dst_v.at[j]], add=True)

        @pl.when(j + 2 < slab)
        def _():
          pltpu.async_copy(table_hbm.at[src_v.at[j + 2]], rows0_v, gsem0)

        pltpu.make_async_copy(table_hbm.at[src_v.at[j + 1]], rows1_v,
                              gsem1).wait()
        pltpu.sync_copy(rows1_v, acc_sh.at[dst_v.at[j + 1]], add=True)

        @pl.when(j + 3 < slab)
        def _():
          pltpu.async_copy(table_hbm.at[src_v.at[j + 3]], rows1_v, gsem1)

        return carry2

      lax.fori_loop(0, slab // 2, pair, 0)
      return carry

    lax.fori_loop(0, n_loads, half, 0)
    plsc.subcore_barrier()

    pltpu.sync_copy(acc_sh.at[pl.ds(s * ACCZ_PER_TILE, ACCZ_PER_TILE)],
                    out_hbm.at[pl.ds(c * ACC_ROWS + s * ACCZ_PER_TILE,
                                     ACCZ_PER_TILE)])

  return pl.kernel(
      body,
      out_type=jax.ShapeDtypeStruct((2 * ACC_ROWS, width), jnp.float32),
      mesh=mesh,
      scratch_types=(
          pltpu.VMEM((slab, CHUNK), jnp.int32),
          pltpu.VMEM((slab, CHUNK), jnp.int32),
          pltpu.VMEM((CHUNK, width), jnp.float32),
          pltpu.VMEM((CHUNK, width), jnp.float32),
          pltpu.VMEM_SHARED((ACC_ROWS, width), jnp.float32),
          pltpu.SemaphoreType.DMA,
          pltpu.SemaphoreType.DMA,
          pltpu.SemaphoreType.DMA,
          pltpu.SemaphoreType.DMA,
      ),
  )


_edge_agg_l1 = _make_edge_agg(CH_PER_TILE // 2, True)
_edge_agg_h = _make_edge_agg(CH_PER_TILE, False)


def _deg_body(dst_hbm, deg_hbm, dst_v, ones_v, dacc_sh):
  c = lax.axis_index("c")
  s = lax.axis_index("s")
  w = c * NS + s

  pltpu.sync_copy(dst_hbm.at[w], dst_v)
  _zero_vmem_2d(ones_v, CHUNK, 128)
  for b in range(ACCZ_PER_TILE // CHUNK):
    pltpu.sync_copy(ones_v, dacc_sh.at[pl.ds(s * ACCZ_PER_TILE + b * CHUNK,
                                             CHUNK)])
  _fill_ones_2d(ones_v, CHUNK, 16)
  plsc.subcore_barrier()

  def chunk(j, carry):
    pltpu.sync_copy(ones_v, dacc_sh.at[dst_v.at[j]], add=True)
    return carry

  lax.fori_loop(0, CH_PER_TILE // 2, chunk, 0)
  plsc.subcore_barrier()
  pltpu.sync_copy(dacc_sh.at[pl.ds(s * ACCZ_PER_TILE, ACCZ_PER_TILE)],
                  deg_hbm.at[pl.ds(c * ACC_ROWS + s * ACCZ_PER_TILE,
                                   ACCZ_PER_TILE)])


_deg_k = pl.kernel(
    _deg_body,
    out_type=jax.ShapeDtypeStruct((2 * ACC_ROWS, 128), jnp.float32),
    mesh=plsc.VectorSubcoreMesh(core_axis_name="c", subcore_axis_name="s"),
    scratch_types=(
        pltpu.VMEM((CH_PER_TILE // 2, CHUNK), jnp.int32),
        pltpu.VMEM((CHUNK, 128), jnp.float32),
        pltpu.VMEM_SHARED((ACC_ROWS, 128), jnp.float32),
    ),
)


def _pool_body(h_hbm, batch_hbm, ridx_hbm, out_hbm, bidx_v, ridx_v, rows_v,
               acc_sh, gsem):
  c = lax.axis_index("c")
  s = lax.axis_index("s")
  w = c * NS + s

  pltpu.sync_copy(batch_hbm.at[s], bidx_v)
  pltpu.sync_copy(ridx_hbm.at[w], ridx_v)
  _zero_vmem_2d(rows_v, G_PER_TILE, D_H // NC)
  pltpu.sync_copy(rows_v.at[pl.ds(0, G_PER_TILE)],
                  acc_sh.at[pl.ds(s * G_PER_TILE, G_PER_TILE)])
  plsc.subcore_barrier()

  def chunk(j, carry):
    pltpu.async_copy(h_hbm.at[ridx_v.at[j]], rows_v, gsem).wait()
    pltpu.sync_copy(rows_v, acc_sh.at[bidx_v.at[j]], add=True)
    return carry

  lax.fori_loop(0, P_CH, chunk, 0)
  plsc.subcore_barrier()
  pltpu.sync_copy(acc_sh.at[pl.ds(s * G_PER_TILE, G_PER_TILE)],
                  out_hbm.at[pl.ds(c * G + s * G_PER_TILE, G_PER_TILE)])


_pool_k = pl.kernel(
    _pool_body,
    out_type=jax.ShapeDtypeStruct((2 * G, D_H // NC), jnp.float32),
    mesh=plsc.VectorSubcoreMesh(core_axis_name="c", subcore_axis_name="s"),
    scratch_types=(
        pltpu.VMEM((P_CH, P_CHUNK), jnp.int32),
        pltpu.VMEM((P_CH, P_CHUNK), jnp.int32),
        pltpu.VMEM((P_CHUNK, D_H // NC), jnp.float32),
        pltpu.VMEM_SHARED((G, D_H // NC), jnp.float32),
        pltpu.SemaphoreType.DMA,
    ),
)


_ROW_BLK = 5000
_N_RB = N // _ROW_BLK


def _tc_layer(a_parts, wl_parts, x_parts, wr_parts, d0, d1, b2d):
  na, nx = len(a_parts), len(x_parts)

  def body(*refs):
    a_refs = refs[:na]
    wl_refs = refs[na:2 * na]
    x_refs = refs[2 * na:2 * na + nx]
    wr_refs = refs[2 * na + nx:2 * na + 2 * nx]
    d0r, d1r, br = refs[-4], refs[-3], refs[-2]
    out = refs[-1]
    deg = d0r[:, 0:1] + d1r[:, 0:1]
    scale = 1.0 / jnp.maximum(deg, 1.0)
    m = sum(jnp.dot(a[...], w[...], preferred_element_type=jnp.float32)
            for a, w in zip(a_refs, wl_refs))
    r = sum(jnp.dot(x[...], w[...], preferred_element_type=jnp.float32)
            for x, w in zip(x_refs, wr_refs))
    brow = br[pl.ds(pl.program_id(1), 1), :]
    out[...] = jnp.maximum(m * scale + brow + r, 0.0)

  row_spec = lambda width: pl.BlockSpec((_ROW_BLK, width),
                                        lambda i, c: (i, 0))
  col_spec = lambda width: pl.BlockSpec((width, 128), lambda i, c: (0, c))
  in_specs = (
      [row_spec(a.shape[1]) for a in a_parts] +
      [col_spec(w.shape[0]) for w in wl_parts] +
      [row_spec(x.shape[1]) for x in x_parts] +
      [col_spec(w.shape[0]) for w in wr_parts] +
      [row_spec(d0.shape[1]), row_spec(d1.shape[1]),
       pl.BlockSpec((2, 128), lambda i, c: (0, 0))]
  )
  return pl.pallas_call(
      body,
      grid=(_N_RB, 2),
      in_specs=in_specs,
      out_specs=pl.BlockSpec((_ROW_BLK, 128), lambda i, c: (c * _N_RB + i, 0)),
      out_shape=jax.ShapeDtypeStruct((2 * N, 128), jnp.float32),
  )(*a_parts, *wl_parts, *x_parts, *wr_parts, d0, d1, b2d)


def _head_body(p0, p1, w1a, w1b, b1, w2, b2, out):
  h = (jnp.dot(p0[...], w1a[...], preferred_element_type=jnp.float32) +
       jnp.dot(p1[...], w1b[...], preferred_element_type=jnp.float32) +
       b1[...])
  h = jnp.maximum(h, 0.0)
  out[...] = jnp.dot(h, w2[...], preferred_element_type=jnp.float32) + b2[...]


def _tc_head(p0, p1, w1a, w1b, b1, w2, b2):
  return pl.pallas_call(
      _head_body,
      out_shape=jax.ShapeDtypeStruct((G, D_OUT), jnp.float32),
  )(p0, p1, w1a, w1b, b1, w2, b2)


def kernel(x, edge_index, batch, Wl1, bl1, Wr1, Wl2, bl2, Wr2, Wl3, bl3, Wr3,
           W_lin1, b_lin1, W_lin2, b_lin2):
  src = edge_index[0].astype(jnp.int32)
  dst = edge_index[1].astype(jnp.int32)

  pad = E_PAD - E
  src_p = jnp.concatenate([src, jnp.zeros((pad,), jnp.int32)])
  dst_p = jnp.concatenate([dst, jnp.full((pad,), N, jnp.int32)])
  src_l1 = src_p.reshape(NC * NS, CH_PER_TILE // 2, CHUNK)
  src_l1 = jnp.concatenate([src_l1[:NS], src_l1[NS:] + N], axis=0)
  dst_l1 = dst_p.reshape(NC * NS, CH_PER_TILE // 2, CHUNK)
  src_r = src_p.reshape(NS, CH_PER_TILE, CHUNK)
  srcw = jnp.concatenate([src_r, src_r + N], axis=0)
  dst_r = dst_p.reshape(NS, CH_PER_TILE, CHUNK)
  batch_r = batch.astype(jnp.int32).reshape(NS, P_CH, P_CHUNK)
  ridx = jnp.arange(N, dtype=jnp.int32).reshape(NS, P_CH, P_CHUNK)
  ridx_w = jnp.concatenate([ridx, ridx + N], axis=0)

  agg1 = _edge_agg_l1(jnp.concatenate([x, x], axis=0), src_l1, dst_l1)
  degp = _deg_k(dst_l1)
  d0, d1 = degp[:N], degp[ACC_ROWS:ACC_ROWS + N]
  h = _tc_layer([agg1[:N], agg1[ACC_ROWS:ACC_ROWS + N]], [Wl1.T, Wl1.T],
                [x], [Wr1.T], d0, d1, bl1.reshape(2, 128))

  for Wl, bl, Wr in ((Wl2, bl2, Wr2), (Wl3, bl3, Wr3)):
    agg = _edge_agg_h(h, srcw, dst_r)
    h = _tc_layer([agg[:N], agg[ACC_ROWS:ACC_ROWS + N]],
                  [Wl.T[:128], Wl.T[128:]],
                  [h[:N], h[N:]],
                  [Wr.T[:128], Wr.T[128:]],
                  d0, d1, bl.reshape(2, 128))

  pooled = _pool_k(h, batch_r, ridx_w)
  out = _tc_head(pooled[:G], pooled[G:],
                 W_lin1.T[:128], W_lin1.T[128:], b_lin1.reshape(1, D_H),
                 W_lin2.T, b_lin2.reshape(1, D_OUT))
  return out
